# baseline (speedup 1.0000x reference)
import jax
import jax.numpy as jnp
from jax.experimental import pallas as pl
from jax.experimental.pallas import tpu as pltpu


_BLOCK = 1024


def _body(x_ref, hp_ref, sel_ref, out_ref):
    k = hp_ref[0, :]
    m = hp_ref[1, :]
    km = k * m
    a = 0.5 * k
    const = 0.25 * jnp.sum(km * m)
    x = x_ref[...]
    t = x * (a[None, :] * x - km[None, :])
    tt = t.T
    e2 = jax.lax.dot_general(
        sel_ref[...], tt, (((1,), (0,)), ((), ())),
        preferred_element_type=jnp.float32,
    )
    out_ref[...] = e2 + const


def kernel(in_feat, harmonic_parameters):
    n, f = in_feat.shape
    n2, f2 = n // 2, f * 2
    xr = in_feat.reshape(n2, f2)
    hp2 = jnp.tile(harmonic_parameters, (1, 2))
    half = (jax.lax.iota(jnp.int32, f2) >= f).astype(jnp.float32)
    sel = jnp.stack([1.0 - half, half], axis=0)
    out = pl.pallas_call(
        _body,
        grid=(n2 // _BLOCK,),
        in_specs=[
            pl.BlockSpec((_BLOCK, f2), lambda i: (i, 0)),
            pl.BlockSpec((2, f2), lambda i: (0, 0)),
            pl.BlockSpec((2, f2), lambda i: (0, 0)),
        ],
        out_specs=pl.BlockSpec((2, _BLOCK), lambda i: (0, i)),
        out_shape=jax.ShapeDtypeStruct((2, n2), jnp.float32),
        compiler_params=pltpu.CompilerParams(
            allow_input_fusion=[True, False, False],
            dimension_semantics=("arbitrary",),
        ),
    )(xr, hp2, sel)
    return out.T.reshape(n, 1)


# speedup vs baseline: 2.2206x; 2.2206x over previous
import jax
import jax.numpy as jnp
from jax.experimental import pallas as pl
from jax.experimental.pallas import tpu as pltpu


_NCHUNK = 16


def _body(x_hbm, hp_ref, out_ref, x_vmem, sems):
    nv, fv = x_vmem.shape
    rows = nv // _NCHUNK

    def copy(c):
        return pltpu.make_async_copy(
            x_hbm.at[pl.ds(c * rows, rows), :],
            x_vmem.at[pl.ds(c * rows, rows), :],
            sems.at[c],
        )

    for c in range(_NCHUNK):
        copy(c).start()

    k = hp_ref[0, :]
    m = hp_ref[1, :]
    km = k * m
    a = 0.5 * k
    const = 0.5 * jnp.sum(km * m)
    ones = jnp.ones((1, fv), dtype=jnp.float32)
    for c in range(_NCHUNK):
        copy(c).wait()
        x = x_vmem[pl.ds(c * rows, rows), :]
        t = x * (a[None, :] * x - km[None, :])
        tt = t.T
        e = jax.lax.dot_general(
            ones, tt, (((1,), (0,)), ((), ())),
            preferred_element_type=jnp.float32,
        )
        out_ref[:, pl.ds(c * rows, rows)] = e + const


def kernel(in_feat, harmonic_parameters):
    n, f = in_feat.shape
    out = pl.pallas_call(
        _body,
        in_specs=[
            pl.BlockSpec(memory_space=pltpu.MemorySpace.HBM),
            pl.BlockSpec((2, f), lambda: (0, 0)),
        ],
        out_specs=pl.BlockSpec((1, n), lambda: (0, 0)),
        out_shape=jax.ShapeDtypeStruct((1, n), jnp.float32),
        scratch_shapes=[
            pltpu.VMEM((n, f), jnp.float32),
            pltpu.SemaphoreType.DMA((_NCHUNK,)),
        ],
        grid=(),
    )(in_feat, harmonic_parameters)
    return out.reshape(n, 1)
